# Initial kernel scaffold; baseline (speedup 1.0000x reference)
import jax
import jax.numpy as jnp
from jax.experimental import pallas as pl


def kernel(edge_index, params):
    raise NotImplementedError("write your pallas kernel here")


# speedup vs baseline: 23.3774x; 23.3774x over previous
import functools

import jax
import jax.numpy as jnp
from jax import lax
from jax.experimental import pallas as pl
from jax.experimental.pallas import tpu as pltpu
from jax.experimental.pallas import tpu_sc as plsc

N_NODES = 100000
NP = 100352
NPK = NP * 16 // 128
NE = 1600000
EP = 1638400
EPK = EP * 16 // 128
NW = 32
ROWS = EP // NW // 128
CH = 16
NCH = ROWS // CH
SLICE = NP // 16
F = 16

_mesh = plsc.VectorSubcoreMesh(core_axis_name="c", subcore_axis_name="s")
_sc_params = pltpu.CompilerParams(use_tc_tiling_on_sc=False)


def _sc_deg_body(src_hbm, part_hbm, src_v, ones_v, acc_sh, sem):
    cid = lax.axis_index("c")
    sid = lax.axis_index("s")
    wid = cid * 16 + sid

    def zrow(i, _):
        ones_v[i, :] = jnp.zeros((F,), jnp.float32)
        return 0
    lax.fori_loop(0, 128, zrow, 0)

    def zslice(c, _):
        pltpu.sync_copy(ones_v, acc_sh.at[pl.ds(sid * SLICE + c * 128, 128)])
        return 0
    lax.fori_loop(0, SLICE // 128, zslice, 0)

    def refill(i, _):
        ones_v[i, :] = jnp.full((F,), 1.0, jnp.float32)
        return 0
    lax.fori_loop(0, 128, refill, 0)
    plsc.subcore_barrier()

    def chunk(c, _):
        pltpu.sync_copy(src_hbm.at[wid, pl.ds(c * CH, CH)], src_v)

        def step(j, _):
            pltpu.sync_copy(ones_v, acc_sh.at[src_v.at[j]], add=True)
            return 0
        lax.fori_loop(0, CH, step, 0)
        return 0
    lax.fori_loop(0, NCH, chunk, 0)
    plsc.subcore_barrier()
    pltpu.sync_copy(acc_sh.at[pl.ds(sid * SLICE, SLICE)],
                    part_hbm.at[cid, pl.ds(sid * SLICE, SLICE)])


_sc_deg = pl.kernel(
    _sc_deg_body,
    out_type=jax.ShapeDtypeStruct((2, NP, F), jnp.float32),
    mesh=_mesh,
    scratch_types=[
        pltpu.VMEM((CH, 128), jnp.int32),
        pltpu.VMEM((128, F), jnp.float32),
        pltpu.VMEM_SHARED((NP, F), jnp.float32),
        pltpu.SemaphoreType.DMA,
    ],
    compiler_params=_sc_params,
)


def _sc_prop_body(y_hbm, src_hbm, dst_hbm, part_hbm,
                  src_v, dst_v, rows_v, zrow_v, acc_sh, sem):
    cid = lax.axis_index("c")
    sid = lax.axis_index("s")
    wid = cid * 16 + sid

    def zrow(i, _):
        zrow_v[i, :] = jnp.zeros((F,), jnp.float32)
        return 0
    lax.fori_loop(0, 128, zrow, 0)

    def zslice(c, _):
        pltpu.sync_copy(zrow_v, acc_sh.at[pl.ds(sid * SLICE + c * 128, 128)])
        return 0
    lax.fori_loop(0, SLICE // 128, zslice, 0)
    plsc.subcore_barrier()

    def chunk(c, _):
        pltpu.sync_copy(src_hbm.at[wid, pl.ds(c * CH, CH)], src_v)
        pltpu.sync_copy(dst_hbm.at[wid, pl.ds(c * CH, CH)], dst_v)

        def step(j, _):
            pltpu.async_copy(y_hbm.at[src_v.at[j]], rows_v, sem).wait()
            pltpu.sync_copy(rows_v, acc_sh.at[dst_v.at[j]], add=True)
            return 0
        lax.fori_loop(0, CH, step, 0)
        return 0
    lax.fori_loop(0, NCH, chunk, 0)
    plsc.subcore_barrier()
    pltpu.sync_copy(acc_sh.at[pl.ds(sid * SLICE, SLICE)],
                    part_hbm.at[cid, pl.ds(sid * SLICE, SLICE)])


_sc_prop = pl.kernel(
    _sc_prop_body,
    out_type=jax.ShapeDtypeStruct((2, NP, F), jnp.float32),
    mesh=_mesh,
    scratch_types=[
        pltpu.VMEM((CH, 128), jnp.int32),
        pltpu.VMEM((CH, 128), jnp.int32),
        pltpu.VMEM((128, F), jnp.float32),
        pltpu.VMEM((128, F), jnp.float32),
        pltpu.VMEM_SHARED((NP, F), jnp.float32),
        pltpu.SemaphoreType.DMA,
    ],
    compiler_params=_sc_params,
)


def _sc_gather_body(c_hbm, src_hbm, dst_hbm, cs_hbm, cd_hbm,
                    src_v, dst_v, rows_v, rowd_v, sem_s, sem_d):
    cid = lax.axis_index("c")
    sid = lax.axis_index("s")
    wid = cid * 16 + sid
    base = wid * (ROWS * 128)

    def chunk(c, _):
        pltpu.sync_copy(src_hbm.at[wid, pl.ds(c * CH, CH)], src_v)
        pltpu.sync_copy(dst_hbm.at[wid, pl.ds(c * CH, CH)], dst_v)

        def step(j, _):
            cps = pltpu.async_copy(c_hbm.at[src_v.at[j]], rows_v, sem_s)
            cpd = pltpu.async_copy(c_hbm.at[dst_v.at[j]], rowd_v, sem_d)
            cps.wait()
            pltpu.sync_copy(
                rows_v, cs_hbm.at[pl.ds(base + (c * CH + j) * 128, 128)])
            cpd.wait()
            pltpu.sync_copy(
                rowd_v, cd_hbm.at[pl.ds(base + (c * CH + j) * 128, 128)])
            return 0
        lax.fori_loop(0, CH, step, 0)
        return 0
    lax.fori_loop(0, NCH, chunk, 0)


_sc_gather = pl.kernel(
    _sc_gather_body,
    out_type=(jax.ShapeDtypeStruct((EP, F), jnp.float32),
              jax.ShapeDtypeStruct((EP, F), jnp.float32)),
    mesh=_mesh,
    scratch_types=[
        pltpu.VMEM((CH, 128), jnp.int32),
        pltpu.VMEM((CH, 128), jnp.int32),
        pltpu.VMEM((128, F), jnp.float32),
        pltpu.VMEM((128, F), jnp.float32),
        pltpu.SemaphoreType.DMA,
        pltpu.SemaphoreType.DMA,
    ],
    compiler_params=_sc_params,
)


_NBLK = 49


def _tc_prep_body(part_ref, x_ref, d16_ref, y0_ref):
    d = part_ref[0] + part_ref[1]
    d16 = jnp.where(d > 0, lax.rsqrt(jnp.maximum(d, 1e-12)), 0.0)
    d16_ref[...] = d16
    y0_ref[...] = d16 * x_ref[...]


def _tc_prep(part, x_pack):
    return pl.pallas_call(
        _tc_prep_body,
        grid=(_NBLK,),
        in_specs=[
            pl.BlockSpec((2, 256, 128), lambda i: (0, i, 0)),
            pl.BlockSpec((256, 128), lambda i: (i, 0)),
        ],
        out_specs=[
            pl.BlockSpec((256, 128), lambda i: (i, 0)),
            pl.BlockSpec((256, 128), lambda i: (i, 0)),
        ],
        out_shape=[jax.ShapeDtypeStruct((NPK, 128), jnp.float32),
                   jax.ShapeDtypeStruct((NPK, 128), jnp.float32)],
    )(part, x_pack)


def _tc_comb_body(alpha, beta, a_ref, d16_ref, tprev_ref, t_ref, y_ref):
    s = a_ref[0] + a_ref[1]
    d16 = d16_ref[...]
    t = (-alpha) * d16 * s - beta * tprev_ref[...]
    t_ref[...] = t
    y_ref[...] = d16 * t


def _tc_comb(alpha, beta, a, d16, tprev):
    return pl.pallas_call(
        functools.partial(_tc_comb_body, alpha, beta),
        grid=(_NBLK,),
        in_specs=[
            pl.BlockSpec((2, 256, 128), lambda i: (0, i, 0)),
            pl.BlockSpec((256, 128), lambda i: (i, 0)),
            pl.BlockSpec((256, 128), lambda i: (i, 0)),
        ],
        out_specs=[
            pl.BlockSpec((256, 128), lambda i: (i, 0)),
            pl.BlockSpec((256, 128), lambda i: (i, 0)),
        ],
        out_shape=[jax.ShapeDtypeStruct((NPK, 128), jnp.float32),
                   jax.ShapeDtypeStruct((NPK, 128), jnp.float32)],
    )(a, d16, tprev)


def _tc_gates_body(t0, t1, t2, t3, t4, wz, wh, wc, bz, bh, c_ref):
    ts = (t0[...], t1[...], t2[...], t3[...], t4[...])
    sz = jnp.broadcast_to(bz[...], (256, 128))
    sh = jnp.broadcast_to(bh[...], (256, 128))
    for k in range(5):
        sz = sz + jnp.dot(ts[k], wz[k], preferred_element_type=jnp.float32)
        sh = sh + jnp.dot(ts[k], wh[k], preferred_element_type=jnp.float32)
    z = jax.nn.sigmoid(sz)
    ht = jnp.tanh(sh)
    h = jax.nn.relu((1.0 - z) * ht)
    c_ref[...] = jnp.dot(h, wc[...], preferred_element_type=jnp.float32)


def _tc_gates(t0, t1, t2, t3, t4, wz, wh, wc, bz, bh):
    blk = pl.BlockSpec((256, 128), lambda i: (i, 0))
    wspec3 = pl.BlockSpec((5, 128, 128), lambda i: (0, 0, 0))
    wspec2 = pl.BlockSpec((128, 128), lambda i: (0, 0))
    bspec = pl.BlockSpec((1, 128), lambda i: (0, 0))
    return pl.pallas_call(
        _tc_gates_body,
        grid=(_NBLK,),
        in_specs=[blk, blk, blk, blk, blk, wspec3, wspec3, wspec2, bspec, bspec],
        out_specs=blk,
        out_shape=jax.ShapeDtypeStruct((NPK, 128), jnp.float32),
    )(t0, t1, t2, t3, t4, wz, wh, wc, bz, bh)


_EBLK = EPK // 512


def _tc_mlp_body(cs, cd, pa, pb, w14, w5, bt, out_ref):
    x = jnp.dot(cs[...], pa[...], preferred_element_type=jnp.float32)
    x = x + jnp.dot(cd[...], pb[...], preferred_element_type=jnp.float32)
    x = jax.nn.relu(x + bt[0])
    for l in range(4):
        x = jax.nn.relu(
            jnp.dot(x, w14[l], preferred_element_type=jnp.float32) + bt[l + 1])
    x = jax.nn.relu(jnp.dot(x, w5[...], preferred_element_type=jnp.float32)
                    + bt[5])
    m = x
    for sh in (1, 2, 126, 127):
        m = jnp.maximum(m, pltpu.roll(x, sh, 1))
    lane = lax.broadcasted_iota(jnp.int32, (512, 128), 1)
    e = jnp.where(lane % 16 < 3, jnp.exp(x - m), 0.0)
    s = e
    for sh in (1, 2, 126, 127):
        s = s + pltpu.roll(e, sh, 1)
    s = jnp.where(s > 0, s, 1.0)
    out_ref[...] = e / s


def _tc_mlp(cs_pack, cd_pack, pa, pb, w14, w5, bt):
    blk = pl.BlockSpec((512, 128), lambda i: (i, 0))
    wspec3 = pl.BlockSpec((4, 128, 128), lambda i: (0, 0, 0))
    wspec2 = pl.BlockSpec((128, 128), lambda i: (0, 0))
    bspec = pl.BlockSpec((6, 128), lambda i: (0, 0))
    return pl.pallas_call(
        _tc_mlp_body,
        grid=(_EBLK,),
        in_specs=[blk, blk, wspec2, wspec2, wspec3, wspec2, bspec],
        out_specs=blk,
        out_shape=jax.ShapeDtypeStruct((EPK, 128), jnp.float32),
    )(cs_pack, cd_pack, pa, pb, w14, w5, bt)


def _bd8(w16):
    return jnp.kron(jnp.eye(8, dtype=jnp.float32), w16)


def _tile8(v16):
    return jnp.tile(v16, 8).reshape(1, 128)


def kernel(edge_index, params):
    p = params
    src = edge_index[0]
    dst = edge_index[1]
    pad = jnp.full((EP - NE,), N_NODES, jnp.int32)
    src3 = jnp.concatenate([src, pad]).reshape(NW, ROWS, 128)
    dst3 = jnp.concatenate([dst, pad]).reshape(NW, ROWS, 128)

    x_pad = jnp.zeros((NP, F), jnp.float32).at[:N_NODES].set(p["emb"])
    x_pack = x_pad.reshape(NPK, 128)

    wz = jnp.stack([_bd8(p["W_xz"][k]) for k in range(5)])
    wh = jnp.stack([_bd8(p["W_xh"][k]) for k in range(5)])
    w0 = p["lin0_w"]
    wc = _bd8(jnp.concatenate([w0[:16], w0[16:]], axis=1))
    bz = _tile8(p["b_xz"] + p["b_hz"])
    bh = _tile8(p["b_xh"] + p["b_hh"])
    eye8 = jnp.eye(8, dtype=jnp.float32)
    zz = jnp.zeros((8, 8), jnp.float32)
    pa = _bd8(jnp.block([[eye8, zz], [zz, zz]]))
    pb = _bd8(jnp.block([[zz, zz], [eye8, zz]]))
    w14 = jnp.stack([
        _bd8(jnp.zeros((16, 16), jnp.float32).at[:8, :8].set(p["lin%d_w" % j]))
        for j in range(1, 5)])
    w5 = _bd8(jnp.zeros((16, 16), jnp.float32).at[:8, :3].set(p["lin5_w"]))
    bt = jnp.concatenate([
        _tile8(jnp.concatenate([p["lin0_b"], jnp.zeros((8,), jnp.float32)])),
        _tile8(jnp.concatenate([p["lin1_b"], jnp.zeros((8,), jnp.float32)])),
        _tile8(jnp.concatenate([p["lin2_b"], jnp.zeros((8,), jnp.float32)])),
        _tile8(jnp.concatenate([p["lin3_b"], jnp.zeros((8,), jnp.float32)])),
        _tile8(jnp.concatenate([p["lin4_b"], jnp.zeros((8,), jnp.float32)])),
        _tile8(jnp.concatenate([p["lin5_b"], jnp.zeros((13,), jnp.float32)])),
    ], axis=0)

    deg_part = _sc_deg(src3)
    d16, y0 = _tc_prep(deg_part.reshape(2, NPK, 128), x_pack)

    t0 = x_pack
    a = _sc_prop(y0.reshape(NP, F), src3, dst3)
    t1, y1 = _tc_comb(1.0, 0.0, a.reshape(2, NPK, 128), d16, t0)
    a = _sc_prop(y1.reshape(NP, F), src3, dst3)
    t2, y2 = _tc_comb(2.0, 1.0, a.reshape(2, NPK, 128), d16, t0)
    a = _sc_prop(y2.reshape(NP, F), src3, dst3)
    t3, y3 = _tc_comb(2.0, 1.0, a.reshape(2, NPK, 128), d16, t1)
    a = _sc_prop(y3.reshape(NP, F), src3, dst3)
    t4, _ = _tc_comb(2.0, 1.0, a.reshape(2, NPK, 128), d16, t2)

    c = _tc_gates(t0, t1, t2, t3, t4, wz, wh, wc, bz, bh)
    cs, cd = _sc_gather(c.reshape(NP, F), src3, dst3)
    out = _tc_mlp(cs.reshape(EPK, 128), cd.reshape(EPK, 128),
                  pa, pb, w14, w5, bt)
    return out.reshape(EP, F)[:NE, :3]


# speedup vs baseline: 26.4623x; 1.1320x over previous
import functools

import jax
import jax.numpy as jnp
from jax import lax
from jax.experimental import pallas as pl
from jax.experimental.pallas import tpu as pltpu
from jax.experimental.pallas import tpu_sc as plsc

N_NODES = 100000
NP = 100352
NPK = NP * 16 // 128
NE = 1600000
EP = 1638400
EPK = EP * 16 // 128
NW = 32
ROWS = EP // NW // 128
EC = 1024
NCC = ROWS * 128 // EC
CH = 16
NCH = ROWS // CH
SLICE = NP // 16
F = 16

_mesh = plsc.VectorSubcoreMesh(core_axis_name="c", subcore_axis_name="s")
_sc_params = pltpu.CompilerParams(use_tc_tiling_on_sc=False)


def _sc_deg_body(src_hbm, part_hbm, src_v, ones_v, zrow_v, acc_sh, sem):
    cid = lax.axis_index("c")
    sid = lax.axis_index("s")
    wid = cid * 16 + sid

    def zrow(i, _):
        zrow_v[i, :] = jnp.zeros((F,), jnp.float32)
        return 0
    lax.fori_loop(0, 128, zrow, 0)

    def zslice(c, _):
        pltpu.sync_copy(zrow_v, acc_sh.at[pl.ds(sid * SLICE + c * 128, 128)])
        return 0
    lax.fori_loop(0, SLICE // 128, zslice, 0)

    def fill(g, _):
        ones_v[g, :] = jnp.full((F,), 1.0, jnp.float32)
        return 0
    lax.fori_loop(0, EC, fill, 0)
    plsc.subcore_barrier()

    def chunk(c, _):
        pltpu.sync_copy(src_hbm.at[wid, c], src_v)
        pltpu.sync_copy(ones_v, acc_sh.at[src_v], add=True)
        return 0
    lax.fori_loop(0, NCC, chunk, 0)
    plsc.subcore_barrier()
    pltpu.sync_copy(acc_sh.at[pl.ds(sid * SLICE, SLICE)],
                    part_hbm.at[cid, pl.ds(sid * SLICE, SLICE)])


_sc_deg = pl.kernel(
    _sc_deg_body,
    out_type=jax.ShapeDtypeStruct((2, NP, F), jnp.float32),
    mesh=_mesh,
    scratch_types=[
        pltpu.VMEM((EC,), jnp.int32),
        pltpu.VMEM((EC, F), jnp.float32),
        pltpu.VMEM((128, F), jnp.float32),
        pltpu.VMEM_SHARED((NP, F), jnp.float32),
        pltpu.SemaphoreType.DMA,
    ],
    compiler_params=_sc_params,
)


def _sc_prop_body(y_hbm, src_hbm, dst_hbm, part_hbm,
                  src_v, dst_v, rows_v, zrow_v, acc_sh, sem):
    cid = lax.axis_index("c")
    sid = lax.axis_index("s")
    wid = cid * 16 + sid

    def zrow(i, _):
        zrow_v[i, :] = jnp.zeros((F,), jnp.float32)
        return 0
    lax.fori_loop(0, 128, zrow, 0)

    def zslice(c, _):
        pltpu.sync_copy(zrow_v, acc_sh.at[pl.ds(sid * SLICE + c * 128, 128)])
        return 0
    lax.fori_loop(0, SLICE // 128, zslice, 0)
    plsc.subcore_barrier()

    def chunk(c, _):
        pltpu.sync_copy(src_hbm.at[wid, c], src_v)
        pltpu.sync_copy(dst_hbm.at[wid, c], dst_v)
        pltpu.async_copy(y_hbm.at[src_v], rows_v, sem).wait()
        pltpu.sync_copy(rows_v, acc_sh.at[dst_v], add=True)
        return 0
    lax.fori_loop(0, NCC, chunk, 0)
    plsc.subcore_barrier()
    pltpu.sync_copy(acc_sh.at[pl.ds(sid * SLICE, SLICE)],
                    part_hbm.at[cid, pl.ds(sid * SLICE, SLICE)])


_sc_prop = pl.kernel(
    _sc_prop_body,
    out_type=jax.ShapeDtypeStruct((2, NP, F), jnp.float32),
    mesh=_mesh,
    scratch_types=[
        pltpu.VMEM((EC,), jnp.int32),
        pltpu.VMEM((EC,), jnp.int32),
        pltpu.VMEM((EC, F), jnp.float32),
        pltpu.VMEM((128, F), jnp.float32),
        pltpu.VMEM_SHARED((NP, F), jnp.float32),
        pltpu.SemaphoreType.DMA,
    ],
    compiler_params=_sc_params,
)


def _sc_gather_body(c_hbm, src_hbm, dst_hbm, cs_hbm, cd_hbm,
                    src_v, dst_v, rows_v, rowd_v, sem_s, sem_d):
    cid = lax.axis_index("c")
    sid = lax.axis_index("s")
    wid = cid * 16 + sid
    base = wid * (ROWS * 128)

    def chunk(c, _):
        pltpu.sync_copy(src_hbm.at[wid, c], src_v)
        pltpu.sync_copy(dst_hbm.at[wid, c], dst_v)
        cps = pltpu.async_copy(c_hbm.at[src_v], rows_v, sem_s)
        cpd = pltpu.async_copy(c_hbm.at[dst_v], rowd_v, sem_d)
        cps.wait()
        pltpu.sync_copy(rows_v, cs_hbm.at[pl.ds(base + c * (CH * 128), CH * 128)])
        cpd.wait()
        pltpu.sync_copy(rowd_v, cd_hbm.at[pl.ds(base + c * (CH * 128), CH * 128)])
        return 0
    lax.fori_loop(0, NCH, chunk, 0)


_sc_gather = pl.kernel(
    _sc_gather_body,
    out_type=(jax.ShapeDtypeStruct((EP, F), jnp.float32),
              jax.ShapeDtypeStruct((EP, F), jnp.float32)),
    mesh=_mesh,
    scratch_types=[
        pltpu.VMEM((CH * 128,), jnp.int32),
        pltpu.VMEM((CH * 128,), jnp.int32),
        pltpu.VMEM((CH * 128, F), jnp.float32),
        pltpu.VMEM((CH * 128, F), jnp.float32),
        pltpu.SemaphoreType.DMA,
        pltpu.SemaphoreType.DMA,
    ],
    compiler_params=_sc_params,
)


_NBLK = 49


def _tc_prep_body(part_ref, x_ref, d16_ref, y0_ref):
    d = part_ref[0] + part_ref[1]
    d16 = jnp.where(d > 0, lax.rsqrt(jnp.maximum(d, 1e-12)), 0.0)
    d16_ref[...] = d16
    y0_ref[...] = d16 * x_ref[...]


def _tc_prep(part, x_pack):
    return pl.pallas_call(
        _tc_prep_body,
        grid=(_NBLK,),
        in_specs=[
            pl.BlockSpec((2, 256, 128), lambda i: (0, i, 0)),
            pl.BlockSpec((256, 128), lambda i: (i, 0)),
        ],
        out_specs=[
            pl.BlockSpec((256, 128), lambda i: (i, 0)),
            pl.BlockSpec((256, 128), lambda i: (i, 0)),
        ],
        out_shape=[jax.ShapeDtypeStruct((NPK, 128), jnp.float32),
                   jax.ShapeDtypeStruct((NPK, 128), jnp.float32)],
    )(part, x_pack)


def _tc_comb_body(alpha, beta, a_ref, d16_ref, tprev_ref, t_ref, y_ref):
    s = a_ref[0] + a_ref[1]
    d16 = d16_ref[...]
    t = (-alpha) * d16 * s - beta * tprev_ref[...]
    t_ref[...] = t
    y_ref[...] = d16 * t


def _tc_comb(alpha, beta, a, d16, tprev):
    return pl.pallas_call(
        functools.partial(_tc_comb_body, alpha, beta),
        grid=(_NBLK,),
        in_specs=[
            pl.BlockSpec((2, 256, 128), lambda i: (0, i, 0)),
            pl.BlockSpec((256, 128), lambda i: (i, 0)),
            pl.BlockSpec((256, 128), lambda i: (i, 0)),
        ],
        out_specs=[
            pl.BlockSpec((256, 128), lambda i: (i, 0)),
            pl.BlockSpec((256, 128), lambda i: (i, 0)),
        ],
        out_shape=[jax.ShapeDtypeStruct((NPK, 128), jnp.float32),
                   jax.ShapeDtypeStruct((NPK, 128), jnp.float32)],
    )(a, d16, tprev)


def _tc_gates_body(t0, t1, t2, t3, t4, wz, wh, wc, bz, bh, c_ref):
    ts = (t0[...], t1[...], t2[...], t3[...], t4[...])
    sz = jnp.broadcast_to(bz[...], (256, 128))
    sh = jnp.broadcast_to(bh[...], (256, 128))
    for k in range(5):
        sz = sz + jnp.dot(ts[k], wz[k], preferred_element_type=jnp.float32)
        sh = sh + jnp.dot(ts[k], wh[k], preferred_element_type=jnp.float32)
    z = jax.nn.sigmoid(sz)
    ht = jnp.tanh(sh)
    h = jax.nn.relu((1.0 - z) * ht)
    c_ref[...] = jnp.dot(h, wc[...], preferred_element_type=jnp.float32)


def _tc_gates(t0, t1, t2, t3, t4, wz, wh, wc, bz, bh):
    blk = pl.BlockSpec((256, 128), lambda i: (i, 0))
    wspec3 = pl.BlockSpec((5, 128, 128), lambda i: (0, 0, 0))
    wspec2 = pl.BlockSpec((128, 128), lambda i: (0, 0))
    bspec = pl.BlockSpec((1, 128), lambda i: (0, 0))
    return pl.pallas_call(
        _tc_gates_body,
        grid=(_NBLK,),
        in_specs=[blk, blk, blk, blk, blk, wspec3, wspec3, wspec2, bspec, bspec],
        out_specs=blk,
        out_shape=jax.ShapeDtypeStruct((NPK, 128), jnp.float32),
    )(t0, t1, t2, t3, t4, wz, wh, wc, bz, bh)


_EBLK = EPK // 512


def _tc_mlp_body(cs, cd, pa, pb, w14, w5, bt, out_ref):
    x = jnp.dot(cs[...], pa[...], preferred_element_type=jnp.float32)
    x = x + jnp.dot(cd[...], pb[...], preferred_element_type=jnp.float32)
    x = jax.nn.relu(x + bt[0])
    for l in range(4):
        x = jax.nn.relu(
            jnp.dot(x, w14[l], preferred_element_type=jnp.float32) + bt[l + 1])
    x = jax.nn.relu(jnp.dot(x, w5[...], preferred_element_type=jnp.float32)
                    + bt[5])
    m = x
    for sh in (1, 2, 126, 127):
        m = jnp.maximum(m, pltpu.roll(x, sh, 1))
    lane = lax.broadcasted_iota(jnp.int32, (512, 128), 1)
    e = jnp.where(lane % 16 < 3, jnp.exp(x - m), 0.0)
    s = e
    for sh in (1, 2, 126, 127):
        s = s + pltpu.roll(e, sh, 1)
    s = jnp.where(s > 0, s, 1.0)
    out_ref[...] = e / s


def _tc_mlp(cs_pack, cd_pack, pa, pb, w14, w5, bt):
    blk = pl.BlockSpec((512, 128), lambda i: (i, 0))
    wspec3 = pl.BlockSpec((4, 128, 128), lambda i: (0, 0, 0))
    wspec2 = pl.BlockSpec((128, 128), lambda i: (0, 0))
    bspec = pl.BlockSpec((6, 128), lambda i: (0, 0))
    return pl.pallas_call(
        _tc_mlp_body,
        grid=(_EBLK,),
        in_specs=[blk, blk, wspec2, wspec2, wspec3, wspec2, bspec],
        out_specs=blk,
        out_shape=jax.ShapeDtypeStruct((EPK, 128), jnp.float32),
    )(cs_pack, cd_pack, pa, pb, w14, w5, bt)


def _bd8(w16):
    return jnp.kron(jnp.eye(8, dtype=jnp.float32), w16)


def _tile8(v16):
    return jnp.tile(v16, 8).reshape(1, 128)


def kernel(edge_index, params):
    p = params
    src = edge_index[0]
    dst = edge_index[1]
    pad = jnp.full((EP - NE,), N_NODES, jnp.int32)
    srcp = jnp.concatenate([src, pad])
    dstp = jnp.concatenate([dst, pad])
    src3 = srcp.reshape(NW, NCC, EC)
    dst3 = dstp.reshape(NW, NCC, EC)
    src3g = srcp.reshape(NW, NCH, CH * 128)
    dst3g = dstp.reshape(NW, NCH, CH * 128)

    x_pad = jnp.zeros((NP, F), jnp.float32).at[:N_NODES].set(p["emb"])
    x_pack = x_pad.reshape(NPK, 128)

    wz = jnp.stack([_bd8(p["W_xz"][k]) for k in range(5)])
    wh = jnp.stack([_bd8(p["W_xh"][k]) for k in range(5)])
    w0 = p["lin0_w"]
    wc = _bd8(jnp.concatenate([w0[:16], w0[16:]], axis=1))
    bz = _tile8(p["b_xz"] + p["b_hz"])
    bh = _tile8(p["b_xh"] + p["b_hh"])
    eye8 = jnp.eye(8, dtype=jnp.float32)
    zz = jnp.zeros((8, 8), jnp.float32)
    pa = _bd8(jnp.block([[eye8, zz], [zz, zz]]))
    pb = _bd8(jnp.block([[zz, zz], [eye8, zz]]))
    w14 = jnp.stack([
        _bd8(jnp.zeros((16, 16), jnp.float32).at[:8, :8].set(p["lin%d_w" % j]))
        for j in range(1, 5)])
    w5 = _bd8(jnp.zeros((16, 16), jnp.float32).at[:8, :3].set(p["lin5_w"]))
    bt = jnp.concatenate([
        _tile8(jnp.concatenate([p["lin0_b"], jnp.zeros((8,), jnp.float32)])),
        _tile8(jnp.concatenate([p["lin1_b"], jnp.zeros((8,), jnp.float32)])),
        _tile8(jnp.concatenate([p["lin2_b"], jnp.zeros((8,), jnp.float32)])),
        _tile8(jnp.concatenate([p["lin3_b"], jnp.zeros((8,), jnp.float32)])),
        _tile8(jnp.concatenate([p["lin4_b"], jnp.zeros((8,), jnp.float32)])),
        _tile8(jnp.concatenate([p["lin5_b"], jnp.zeros((13,), jnp.float32)])),
    ], axis=0)

    deg_part = _sc_deg(src3)
    d16, y0 = _tc_prep(deg_part.reshape(2, NPK, 128), x_pack)

    t0 = x_pack
    a = _sc_prop(y0.reshape(NP, F), src3, dst3)
    t1, y1 = _tc_comb(1.0, 0.0, a.reshape(2, NPK, 128), d16, t0)
    a = _sc_prop(y1.reshape(NP, F), src3, dst3)
    t2, y2 = _tc_comb(2.0, 1.0, a.reshape(2, NPK, 128), d16, t0)
    a = _sc_prop(y2.reshape(NP, F), src3, dst3)
    t3, y3 = _tc_comb(2.0, 1.0, a.reshape(2, NPK, 128), d16, t1)
    a = _sc_prop(y3.reshape(NP, F), src3, dst3)
    t4, _ = _tc_comb(2.0, 1.0, a.reshape(2, NPK, 128), d16, t2)

    c = _tc_gates(t0, t1, t2, t3, t4, wz, wh, wc, bz, bh)
    cs, cd = _sc_gather(c.reshape(NP, F), src3g, dst3g)
    out = _tc_mlp(cs.reshape(EPK, 128), cd.reshape(EPK, 128),
                  pa, pb, w14, w5, bt)
    return out.reshape(EP, F)[:NE, :3]


# speedup vs baseline: 26.5189x; 1.0021x over previous
import functools

import jax
import jax.numpy as jnp
from jax import lax
from jax.experimental import pallas as pl
from jax.experimental.pallas import tpu as pltpu
from jax.experimental.pallas import tpu_sc as plsc

N_NODES = 100000
NP = 100352
NPK = NP * 16 // 128
NE = 1600000
EP = 1638400
EPK = EP * 16 // 128
NW = 32
ROWS = EP // NW // 128
EC = 512
NCC = ROWS * 128 // EC
EG = 1024
NCH = ROWS * 128 // EG
SLICE = NP // 16
F = 16

_mesh = plsc.VectorSubcoreMesh(core_axis_name="c", subcore_axis_name="s")
_sc_params = pltpu.CompilerParams(use_tc_tiling_on_sc=False)


def _sc_deg_body(src_hbm, part_hbm, src_v, srcb_v, ones_v, zrow_v, acc_sh,
                 sem, semb):
    cid = lax.axis_index("c")
    sid = lax.axis_index("s")
    wid = cid * 16 + sid

    def zrow(i, _):
        zrow_v[i, :] = jnp.zeros((F,), jnp.float32)
        return 0
    lax.fori_loop(0, 128, zrow, 0)

    def zslice(c, _):
        pltpu.sync_copy(zrow_v, acc_sh.at[pl.ds(sid * SLICE + c * 128, 128)])
        return 0
    lax.fori_loop(0, SLICE // 128, zslice, 0)

    def fill(g, _):
        ones_v[g, :] = jnp.full((F,), 1.0, jnp.float32)
        return 0
    lax.fori_loop(0, EC, fill, 0)
    plsc.subcore_barrier()

    def one_chunk(c, idx_v, ssem):
        @pl.when(c >= 2)
        def _():
            pltpu.make_async_copy(ones_v, acc_sh.at[idx_v], ssem).wait()
        pltpu.sync_copy(src_hbm.at[wid, c], idx_v)
        pltpu.async_copy(ones_v, acc_sh.at[idx_v], ssem, add=True)

    def chunk(c, _):
        @pl.when(c % 2 == 0)
        def _():
            one_chunk(c, src_v, sem)
        @pl.when(c % 2 == 1)
        def _():
            one_chunk(c, srcb_v, semb)
        return 0
    lax.fori_loop(0, NCC, chunk, 0)
    pltpu.make_async_copy(ones_v, acc_sh.at[src_v], sem).wait()
    pltpu.make_async_copy(ones_v, acc_sh.at[srcb_v], semb).wait()
    plsc.subcore_barrier()
    pltpu.sync_copy(acc_sh.at[pl.ds(sid * SLICE, SLICE)],
                    part_hbm.at[cid, pl.ds(sid * SLICE, SLICE)])


_sc_deg = pl.kernel(
    _sc_deg_body,
    out_type=jax.ShapeDtypeStruct((2, NP, F), jnp.float32),
    mesh=_mesh,
    scratch_types=[
        pltpu.VMEM((EC,), jnp.int32),
        pltpu.VMEM((EC,), jnp.int32),
        pltpu.VMEM((EC, F), jnp.float32),
        pltpu.VMEM((128, F), jnp.float32),
        pltpu.VMEM_SHARED((NP, F), jnp.float32),
        pltpu.SemaphoreType.DMA,
        pltpu.SemaphoreType.DMA,
    ],
    compiler_params=_sc_params,
)


def _sc_prop_body(y_hbm, src_hbm, dst_hbm, part_hbm,
                  src_a, dst_a, rows_a, src_b, dst_b, rows_b,
                  zrow_v, acc_sh, gsem_a, ssem_a, gsem_b, ssem_b):
    cid = lax.axis_index("c")
    sid = lax.axis_index("s")
    wid = cid * 16 + sid

    def zrow(i, _):
        zrow_v[i, :] = jnp.zeros((F,), jnp.float32)
        return 0
    lax.fori_loop(0, 128, zrow, 0)

    def zslice(c, _):
        pltpu.sync_copy(zrow_v, acc_sh.at[pl.ds(sid * SLICE + c * 128, 128)])
        return 0
    lax.fori_loop(0, SLICE // 128, zslice, 0)
    plsc.subcore_barrier()

    def one_chunk(c, src_v, dst_v, rows_v, gsem, ssem):
        @pl.when(c >= 2)
        def _():
            pltpu.make_async_copy(rows_v, acc_sh.at[dst_v], ssem).wait()
        pltpu.sync_copy(src_hbm.at[wid, c], src_v)
        pltpu.sync_copy(dst_hbm.at[wid, c], dst_v)
        pltpu.async_copy(y_hbm.at[src_v], rows_v, gsem).wait()
        pltpu.async_copy(rows_v, acc_sh.at[dst_v], ssem, add=True)

    def chunk(c, _):
        @pl.when(c % 2 == 0)
        def _():
            one_chunk(c, src_a, dst_a, rows_a, gsem_a, ssem_a)
        @pl.when(c % 2 == 1)
        def _():
            one_chunk(c, src_b, dst_b, rows_b, gsem_b, ssem_b)
        return 0
    lax.fori_loop(0, NCC, chunk, 0)
    pltpu.make_async_copy(rows_a, acc_sh.at[dst_a], ssem_a).wait()
    pltpu.make_async_copy(rows_b, acc_sh.at[dst_b], ssem_b).wait()
    plsc.subcore_barrier()
    pltpu.sync_copy(acc_sh.at[pl.ds(sid * SLICE, SLICE)],
                    part_hbm.at[cid, pl.ds(sid * SLICE, SLICE)])


_sc_prop = pl.kernel(
    _sc_prop_body,
    out_type=jax.ShapeDtypeStruct((2, NP, F), jnp.float32),
    mesh=_mesh,
    scratch_types=[
        pltpu.VMEM((EC,), jnp.int32),
        pltpu.VMEM((EC,), jnp.int32),
        pltpu.VMEM((EC, F), jnp.float32),
        pltpu.VMEM((EC,), jnp.int32),
        pltpu.VMEM((EC,), jnp.int32),
        pltpu.VMEM((EC, F), jnp.float32),
        pltpu.VMEM((128, F), jnp.float32),
        pltpu.VMEM_SHARED((NP, F), jnp.float32),
        pltpu.SemaphoreType.DMA,
        pltpu.SemaphoreType.DMA,
        pltpu.SemaphoreType.DMA,
        pltpu.SemaphoreType.DMA,
    ],
    compiler_params=_sc_params,
)


def _sc_gather_body(c_hbm, src_hbm, dst_hbm, cs_hbm, cd_hbm,
                    src_a, dst_a, rows_a, rowd_a,
                    src_b, dst_b, rows_b, rowd_b,
                    gs_a, gd_a, ws_a, wd_a, gs_b, gd_b, ws_b, wd_b):
    cid = lax.axis_index("c")
    sid = lax.axis_index("s")
    wid = cid * 16 + sid
    base = wid * (ROWS * 128)

    def one_chunk(c, src_v, dst_v, rows_v, rowd_v, gs, gd, ws, wd):
        @pl.when(c >= 2)
        def _():
            pltpu.make_async_copy(
                rows_v, cs_hbm.at[pl.ds(base, EG)], ws).wait()
            pltpu.make_async_copy(
                rowd_v, cd_hbm.at[pl.ds(base, EG)], wd).wait()
        pltpu.sync_copy(src_hbm.at[wid, c], src_v)
        pltpu.sync_copy(dst_hbm.at[wid, c], dst_v)
        cps = pltpu.async_copy(c_hbm.at[src_v], rows_v, gs)
        cpd = pltpu.async_copy(c_hbm.at[dst_v], rowd_v, gd)
        cps.wait()
        pltpu.async_copy(rows_v, cs_hbm.at[pl.ds(base + c * EG, EG)], ws)
        cpd.wait()
        pltpu.async_copy(rowd_v, cd_hbm.at[pl.ds(base + c * EG, EG)], wd)

    def chunk(c, _):
        @pl.when(c % 2 == 0)
        def _():
            one_chunk(c, src_a, dst_a, rows_a, rowd_a, gs_a, gd_a, ws_a, wd_a)
        @pl.when(c % 2 == 1)
        def _():
            one_chunk(c, src_b, dst_b, rows_b, rowd_b, gs_b, gd_b, ws_b, wd_b)
        return 0
    lax.fori_loop(0, NCH, chunk, 0)
    pltpu.make_async_copy(rows_a, cs_hbm.at[pl.ds(base, EG)], ws_a).wait()
    pltpu.make_async_copy(rowd_a, cd_hbm.at[pl.ds(base, EG)], wd_a).wait()
    pltpu.make_async_copy(rows_b, cs_hbm.at[pl.ds(base, EG)], ws_b).wait()
    pltpu.make_async_copy(rowd_b, cd_hbm.at[pl.ds(base, EG)], wd_b).wait()


_sc_gather = pl.kernel(
    _sc_gather_body,
    out_type=(jax.ShapeDtypeStruct((EP, F), jnp.float32),
              jax.ShapeDtypeStruct((EP, F), jnp.float32)),
    mesh=_mesh,
    scratch_types=(
        [pltpu.VMEM((EG,), jnp.int32), pltpu.VMEM((EG,), jnp.int32),
         pltpu.VMEM((EG, F), jnp.float32), pltpu.VMEM((EG, F), jnp.float32)] * 2
        + [pltpu.SemaphoreType.DMA] * 8),
    compiler_params=_sc_params,
)


_NBLK = 49


def _tc_prep_body(part_ref, x_ref, d16_ref, y0_ref):
    d = part_ref[0] + part_ref[1]
    d16 = jnp.where(d > 0, lax.rsqrt(jnp.maximum(d, 1e-12)), 0.0)
    d16_ref[...] = d16
    y0_ref[...] = d16 * x_ref[...]


def _tc_prep(part, x_pack):
    return pl.pallas_call(
        _tc_prep_body,
        grid=(_NBLK,),
        in_specs=[
            pl.BlockSpec((2, 256, 128), lambda i: (0, i, 0)),
            pl.BlockSpec((256, 128), lambda i: (i, 0)),
        ],
        out_specs=[
            pl.BlockSpec((256, 128), lambda i: (i, 0)),
            pl.BlockSpec((256, 128), lambda i: (i, 0)),
        ],
        out_shape=[jax.ShapeDtypeStruct((NPK, 128), jnp.float32),
                   jax.ShapeDtypeStruct((NPK, 128), jnp.float32)],
    )(part, x_pack)


def _tc_comb_body(alpha, beta, a_ref, d16_ref, tprev_ref, t_ref, y_ref):
    s = a_ref[0] + a_ref[1]
    d16 = d16_ref[...]
    t = (-alpha) * d16 * s - beta * tprev_ref[...]
    t_ref[...] = t
    y_ref[...] = d16 * t


def _tc_comb(alpha, beta, a, d16, tprev):
    return pl.pallas_call(
        functools.partial(_tc_comb_body, alpha, beta),
        grid=(_NBLK,),
        in_specs=[
            pl.BlockSpec((2, 256, 128), lambda i: (0, i, 0)),
            pl.BlockSpec((256, 128), lambda i: (i, 0)),
            pl.BlockSpec((256, 128), lambda i: (i, 0)),
        ],
        out_specs=[
            pl.BlockSpec((256, 128), lambda i: (i, 0)),
            pl.BlockSpec((256, 128), lambda i: (i, 0)),
        ],
        out_shape=[jax.ShapeDtypeStruct((NPK, 128), jnp.float32),
                   jax.ShapeDtypeStruct((NPK, 128), jnp.float32)],
    )(a, d16, tprev)


def _tc_gates_body(t0, t1, t2, t3, t4, wz, wh, wc, bz, bh, c_ref):
    ts = (t0[...], t1[...], t2[...], t3[...], t4[...])
    sz = jnp.broadcast_to(bz[...], (256, 128))
    sh = jnp.broadcast_to(bh[...], (256, 128))
    for k in range(5):
        sz = sz + jnp.dot(ts[k], wz[k], preferred_element_type=jnp.float32)
        sh = sh + jnp.dot(ts[k], wh[k], preferred_element_type=jnp.float32)
    z = jax.nn.sigmoid(sz)
    ht = jnp.tanh(sh)
    h = jax.nn.relu((1.0 - z) * ht)
    c_ref[...] = jnp.dot(h, wc[...], preferred_element_type=jnp.float32)


def _tc_gates(t0, t1, t2, t3, t4, wz, wh, wc, bz, bh):
    blk = pl.BlockSpec((256, 128), lambda i: (i, 0))
    wspec3 = pl.BlockSpec((5, 128, 128), lambda i: (0, 0, 0))
    wspec2 = pl.BlockSpec((128, 128), lambda i: (0, 0))
    bspec = pl.BlockSpec((1, 128), lambda i: (0, 0))
    return pl.pallas_call(
        _tc_gates_body,
        grid=(_NBLK,),
        in_specs=[blk, blk, blk, blk, blk, wspec3, wspec3, wspec2, bspec, bspec],
        out_specs=blk,
        out_shape=jax.ShapeDtypeStruct((NPK, 128), jnp.float32),
    )(t0, t1, t2, t3, t4, wz, wh, wc, bz, bh)


_EBLK = EPK // 512


def _tc_mlp_body(cs, cd, pa, pb, w14, w5, bt, out_ref):
    x = jnp.dot(cs[...], pa[...], preferred_element_type=jnp.float32)
    x = x + jnp.dot(cd[...], pb[...], preferred_element_type=jnp.float32)
    x = jax.nn.relu(x + bt[0])
    for l in range(4):
        x = jax.nn.relu(
            jnp.dot(x, w14[l], preferred_element_type=jnp.float32) + bt[l + 1])
    x = jax.nn.relu(jnp.dot(x, w5[...], preferred_element_type=jnp.float32)
                    + bt[5])
    m = x
    for sh in (1, 2, 126, 127):
        m = jnp.maximum(m, pltpu.roll(x, sh, 1))
    lane = lax.broadcasted_iota(jnp.int32, (512, 128), 1)
    e = jnp.where(lane % 16 < 3, jnp.exp(x - m), 0.0)
    s = e
    for sh in (1, 2, 126, 127):
        s = s + pltpu.roll(e, sh, 1)
    s = jnp.where(s > 0, s, 1.0)
    out_ref[...] = e / s


def _tc_mlp(cs_pack, cd_pack, pa, pb, w14, w5, bt):
    blk = pl.BlockSpec((512, 128), lambda i: (i, 0))
    wspec3 = pl.BlockSpec((4, 128, 128), lambda i: (0, 0, 0))
    wspec2 = pl.BlockSpec((128, 128), lambda i: (0, 0))
    bspec = pl.BlockSpec((6, 128), lambda i: (0, 0))
    return pl.pallas_call(
        _tc_mlp_body,
        grid=(_EBLK,),
        in_specs=[blk, blk, wspec2, wspec2, wspec3, wspec2, bspec],
        out_specs=blk,
        out_shape=jax.ShapeDtypeStruct((EPK, 128), jnp.float32),
    )(cs_pack, cd_pack, pa, pb, w14, w5, bt)


def _bd8(w16):
    return jnp.kron(jnp.eye(8, dtype=jnp.float32), w16)


def _tile8(v16):
    return jnp.tile(v16, 8).reshape(1, 128)


def kernel(edge_index, params):
    p = params
    src = edge_index[0]
    dst = edge_index[1]
    pad = jnp.full((EP - NE,), N_NODES, jnp.int32)
    srcp = jnp.concatenate([src, pad])
    dstp = jnp.concatenate([dst, pad])
    src3 = srcp.reshape(NW, NCC, EC)
    dst3 = dstp.reshape(NW, NCC, EC)
    src3g = srcp.reshape(NW, NCH, EG)
    dst3g = dstp.reshape(NW, NCH, EG)

    x_pad = jnp.zeros((NP, F), jnp.float32).at[:N_NODES].set(p["emb"])
    x_pack = x_pad.reshape(NPK, 128)

    wz = jnp.stack([_bd8(p["W_xz"][k]) for k in range(5)])
    wh = jnp.stack([_bd8(p["W_xh"][k]) for k in range(5)])
    w0 = p["lin0_w"]
    wc = _bd8(jnp.concatenate([w0[:16], w0[16:]], axis=1))
    bz = _tile8(p["b_xz"] + p["b_hz"])
    bh = _tile8(p["b_xh"] + p["b_hh"])
    eye8 = jnp.eye(8, dtype=jnp.float32)
    zz = jnp.zeros((8, 8), jnp.float32)
    pa = _bd8(jnp.block([[eye8, zz], [zz, zz]]))
    pb = _bd8(jnp.block([[zz, zz], [eye8, zz]]))
    w14 = jnp.stack([
        _bd8(jnp.zeros((16, 16), jnp.float32).at[:8, :8].set(p["lin%d_w" % j]))
        for j in range(1, 5)])
    w5 = _bd8(jnp.zeros((16, 16), jnp.float32).at[:8, :3].set(p["lin5_w"]))
    bt = jnp.concatenate([
        _tile8(jnp.concatenate([p["lin0_b"], jnp.zeros((8,), jnp.float32)])),
        _tile8(jnp.concatenate([p["lin1_b"], jnp.zeros((8,), jnp.float32)])),
        _tile8(jnp.concatenate([p["lin2_b"], jnp.zeros((8,), jnp.float32)])),
        _tile8(jnp.concatenate([p["lin3_b"], jnp.zeros((8,), jnp.float32)])),
        _tile8(jnp.concatenate([p["lin4_b"], jnp.zeros((8,), jnp.float32)])),
        _tile8(jnp.concatenate([p["lin5_b"], jnp.zeros((13,), jnp.float32)])),
    ], axis=0)

    deg_part = _sc_deg(src3)
    d16, y0 = _tc_prep(deg_part.reshape(2, NPK, 128), x_pack)

    t0 = x_pack
    a = _sc_prop(y0.reshape(NP, F), src3, dst3)
    t1, y1 = _tc_comb(1.0, 0.0, a.reshape(2, NPK, 128), d16, t0)
    a = _sc_prop(y1.reshape(NP, F), src3, dst3)
    t2, y2 = _tc_comb(2.0, 1.0, a.reshape(2, NPK, 128), d16, t0)
    a = _sc_prop(y2.reshape(NP, F), src3, dst3)
    t3, y3 = _tc_comb(2.0, 1.0, a.reshape(2, NPK, 128), d16, t1)
    a = _sc_prop(y3.reshape(NP, F), src3, dst3)
    t4, _ = _tc_comb(2.0, 1.0, a.reshape(2, NPK, 128), d16, t2)

    c = _tc_gates(t0, t1, t2, t3, t4, wz, wh, wc, bz, bh)
    cs, cd = _sc_gather(c.reshape(NP, F), src3g, dst3g)
    out = _tc_mlp(cs.reshape(EPK, 128), cd.reshape(EPK, 128),
                  pa, pb, w14, w5, bt)
    return out.reshape(EP, F)[:NE, :3]
